# Initial kernel scaffold; baseline (speedup 1.0000x reference)
import jax
import jax.numpy as jnp
from jax.experimental import pallas as pl


def kernel(h, edge_index, edge_labels):
    raise NotImplementedError("write your pallas kernel here")


# speedup vs baseline: 3.6212x; 3.6212x over previous
import functools

import jax
import jax.numpy as jnp
from jax import lax
from jax.experimental import pallas as pl
from jax.experimental.pallas import tpu as pltpu
from jax.experimental.pallas import tpu_sc as plsc

N = 10000
E = 320000
DF = 128
DE = 16
NPAD = 10240
ROWS = E // 128
HBINS = 320256
HPT = HBINS // 16

_mesh = plsc.VectorSubcoreMesh(core_axis_name="c", subcore_axis_name="s")


def _zero_i32(ref, n):
    z = jnp.zeros((16,), jnp.int32)

    def body(i, _):
        ref[pl.ds(i * 16, 16)] = z
        return 0

    lax.fori_loop(0, n // 16, body, 0)


@functools.partial(
    pl.kernel,
    out_type=(
        jax.ShapeDtypeStruct((NPAD, DE), jnp.float32),
        jax.ShapeDtypeStruct((NPAD, DE), jnp.float32),
        jax.ShapeDtypeStruct((NPAD,), jnp.int32),
        jax.ShapeDtypeStruct((NPAD,), jnp.int32),
    ),
    mesh=_mesh,
    compiler_params=pltpu.CompilerParams(use_tc_tiling_on_sc=False, needs_layout_passes=False),
    scratch_types=[
        pltpu.VMEM_SHARED((NPAD, DE), jnp.float32),
        pltpu.VMEM_SHARED((NPAD,), jnp.int32),
        pltpu.VMEM((NPAD // 16, DE), jnp.float32),
        pltpu.VMEM((NPAD // 16,), jnp.int32),
        pltpu.VMEM((128,), jnp.int32),
        pltpu.VMEM((8, 128), jnp.int32),
        pltpu.VMEM((8 * 128, DE), jnp.float32),
        pltpu.SemaphoreType.DMA,
    ],
)
def _scatter(dst_hbm, lab_hbm, sums0_hbm, sums1_hbm, deg0_hbm, deg1_hbm,
             sums_sh, deg_sh, zrow_v, zdeg_v, ones_v, idx_v, lab_v, sem):
    cid = lax.axis_index("c")
    sid = lax.axis_index("s")
    w = sid * 2 + cid
    npt = NPAD // 16

    zf = jnp.zeros((16,), jnp.float32)

    def zrow_body(i, _):
        zrow_v[i, :] = zf
        return 0

    lax.fori_loop(0, npt, zrow_body, 0)
    _zero_i32(zdeg_v, npt)
    one = jnp.ones((16,), jnp.int32)
    for i in range(128 // 16):
        ones_v[pl.ds(i * 16, 16)] = one
    pltpu.sync_copy(zrow_v, sums_sh.at[pl.ds(sid * npt, npt)])
    pltpu.sync_copy(zdeg_v, deg_sh.at[pl.ds(sid * npt, npt)])
    plsc.subcore_barrier()

    def do_rows(row0, nrows):
        pltpu.sync_copy(dst_hbm.at[pl.ds(row0, nrows)],
                        idx_v.at[pl.ds(0, nrows)])
        pltpu.sync_copy(lab_hbm.at[pl.ds(row0 * 128, nrows * 128)],
                        lab_v.at[pl.ds(0, nrows * 128)])
        cps = []
        for j in range(nrows):
            cps.append(pltpu.async_copy(
                lab_v.at[pl.ds(j * 128, 128)],
                sums_sh.at[idx_v.at[j]], sem, add=True))
            cps.append(pltpu.async_copy(
                ones_v, deg_sh.at[idx_v.at[j]], sem, add=True))
        for c in cps:
            c.wait()

    nch = jnp.where(w < 24, 10, 9)
    base_w = jnp.where(w < 24, 80 * w, 1920 + 72 * (w - 24))
    for k in range(10):
        @pl.when(k < nch)
        def _():
            do_rows(base_w + 8 * k, 8)

    @pl.when(w == 31)
    def _():
        do_rows(2496, 4)

    plsc.subcore_barrier()

    @pl.when(cid == 0)
    def _():
        pltpu.sync_copy(sums_sh.at[pl.ds(sid * npt, npt)],
                        sums0_hbm.at[pl.ds(sid * npt, npt)])
        pltpu.sync_copy(deg_sh.at[pl.ds(sid * npt, npt)],
                        deg0_hbm.at[pl.ds(sid * npt, npt)])

    @pl.when(cid == 1)
    def _():
        pltpu.sync_copy(sums_sh.at[pl.ds(sid * npt, npt)],
                        sums1_hbm.at[pl.ds(sid * npt, npt)])
        pltpu.sync_copy(deg_sh.at[pl.ds(sid * npt, npt)],
                        deg1_hbm.at[pl.ds(sid * npt, npt)])


FULL_U = N // 128
TAIL_R = N - FULL_U * 128


@functools.partial(
    pl.kernel,
    out_type=jax.ShapeDtypeStruct((N, DF + DE), jnp.float32),
    mesh=_mesh,
    compiler_params=pltpu.CompilerParams(use_tc_tiling_on_sc=False, needs_layout_passes=False),
    scratch_types=[
        pltpu.VMEM_SHARED((HBINS,), jnp.int32),
        pltpu.VMEM((HPT,), jnp.int32),
        pltpu.VMEM((128,), jnp.int32),
        pltpu.VMEM((NPAD // 16,), jnp.int32),
        pltpu.VMEM((NPAD // 16,), jnp.int32),
        pltpu.VMEM((NPAD // (16 * 128), 128), jnp.int32),
        pltpu.VMEM((1, 128), jnp.int32),
        pltpu.VMEM((128,), jnp.int32),
        pltpu.VMEM((128,), jnp.float32),
        pltpu.VMEM((128, DE), jnp.float32),
        pltpu.VMEM((128, DE), jnp.float32),
        pltpu.VMEM((128, DF), jnp.float32),
        pltpu.VMEM((128, DF + DE), jnp.float32),
        pltpu.SemaphoreType.DMA,
    ],
)
def _finalize(h_hbm, sums0_hbm, sums1_hbm, deg0_hbm, deg1_hbm, out_hbm,
              hist_sh, zb_v, ones_v, d0_v, d1_v, didx_v, du_v,
              bkt_v, bsz_v, s0_v, s1_v, h_v, out_v, sem):
    cid = lax.axis_index("c")
    sid = lax.axis_index("s")
    w = sid * 2 + cid
    npt = NPAD // 16

    _zero_i32(zb_v, HPT)
    one = jnp.ones((16,), jnp.int32)
    for i in range(128 // 16):
        ones_v[pl.ds(i * 16, 16)] = one
    pltpu.sync_copy(zb_v, hist_sh.at[pl.ds(sid * HPT, HPT)])
    plsc.subcore_barrier()

    pltpu.sync_copy(deg0_hbm.at[pl.ds(sid * npt, npt)], d0_v)
    pltpu.sync_copy(deg1_hbm.at[pl.ds(sid * npt, npt)], d1_v)
    nrow = npt // 128

    def deg_body(i, _):
        r = i // 8
        l = i % 8
        didx_v[r, pl.ds(l * 16, 16)] = (
            d0_v[pl.ds(i * 16, 16)] + d1_v[pl.ds(i * 16, 16)])
        return 0

    lax.fori_loop(0, npt // 16, deg_body, 0)
    cps = [pltpu.async_copy(ones_v, hist_sh.at[didx_v.at[r]], sem, add=True)
           for r in range(nrow)]
    for c in cps:
        c.wait()
    plsc.subcore_barrier()

    def elu(x):
        return jnp.where(x > 0, x, jnp.exp(x) - 1.0)

    for k in range(3):
        u = w + 32 * k

        @pl.when(u <= FULL_U)
        def _():
            base = u * 128
            pltpu.sync_copy(deg0_hbm.at[pl.ds(base, 128)], bkt_v)
            pltpu.sync_copy(deg1_hbm.at[pl.ds(base, 128)], du_v.at[0])
            for i in range(8):
                du_v[0, pl.ds(i * 16, 16)] = (
                    bkt_v[pl.ds(i * 16, 16)] + du_v[0, pl.ds(i * 16, 16)])
            pltpu.sync_copy(hist_sh.at[du_v.at[0]], bkt_v)
            for i in range(8):
                b = bkt_v[pl.ds(i * 16, 16)].astype(jnp.float32)
                bsz_v[pl.ds(i * 16, 16)] = jnp.maximum(b, 1.0)
            pltpu.sync_copy(sums0_hbm.at[pl.ds(base, 128)], s0_v)
            pltpu.sync_copy(sums1_hbm.at[pl.ds(base, 128)], s1_v)

            @pl.when(u < FULL_U)
            def _():
                pltpu.sync_copy(h_hbm.at[pl.ds(base, 128)], h_v)

            @pl.when(u == FULL_U)
            def _():
                pltpu.sync_copy(h_hbm.at[pl.ds(FULL_U * 128, TAIL_R)],
                                h_v.at[pl.ds(0, TAIL_R)])

            def row_body(i, _):
                s = s0_v[i, :] + s1_v[i, :]
                bs = plsc.load_gather(
                    bsz_v, [jnp.full((16,), i, dtype=jnp.int32)])
                out_v[i, pl.ds(DF, DE)] = elu(s / bs)
                for j in range(DF // 16):
                    x = h_v[i, pl.ds(j * 16, 16)]
                    out_v[i, pl.ds(j * 16, 16)] = elu(x)
                return 0

            lax.fori_loop(0, 128, row_body, 0)

            @pl.when(u < FULL_U)
            def _():
                pltpu.sync_copy(out_v, out_hbm.at[pl.ds(base, 128)])

            @pl.when(u == FULL_U)
            def _():
                pltpu.sync_copy(out_v.at[pl.ds(0, TAIL_R)],
                                out_hbm.at[pl.ds(FULL_U * 128, TAIL_R)])


def kernel(h, edge_index, edge_labels):
    dst2d = edge_index[1].astype(jnp.int32).reshape(ROWS, 128)
    sums0, sums1, deg0, deg1 = _scatter(dst2d, edge_labels)
    return _finalize(h, sums0, sums1, deg0, deg1)


# speedup vs baseline: 3.8009x; 1.0496x over previous
import functools

import jax
import jax.numpy as jnp
from jax import lax
from jax.experimental import pallas as pl
from jax.experimental.pallas import tpu as pltpu
from jax.experimental.pallas import tpu_sc as plsc

N = 10000
E = 320000
DF = 128
DE = 16
NPAD = 10240
EPW = E // 32
NCH = 5
IW = 125
CR = 16
CE = IW * CR
HBINS = 320256
HPT = HBINS // 16

_mesh = plsc.VectorSubcoreMesh(core_axis_name="c", subcore_axis_name="s")


def _zero_i32(ref, n):
    z = jnp.zeros((16,), jnp.int32)

    def body(i, _):
        ref[pl.ds(i * 16, 16)] = z
        return 0

    lax.fori_loop(0, n // 16, body, 0)


@functools.partial(
    pl.kernel,
    out_type=(
        jax.ShapeDtypeStruct((NPAD, DE), jnp.float32),
        jax.ShapeDtypeStruct((NPAD, DE), jnp.float32),
        jax.ShapeDtypeStruct((NPAD,), jnp.int32),
        jax.ShapeDtypeStruct((NPAD,), jnp.int32),
    ),
    mesh=_mesh,
    compiler_params=pltpu.CompilerParams(use_tc_tiling_on_sc=False, needs_layout_passes=False),
    scratch_types=[
        pltpu.VMEM_SHARED((NPAD, DE), jnp.float32),
        pltpu.VMEM_SHARED((NPAD,), jnp.int32),
        pltpu.VMEM((NPAD // 16, DE), jnp.float32),
        pltpu.VMEM((NPAD // 16,), jnp.int32),
        pltpu.VMEM((IW,), jnp.int32),
        pltpu.VMEM((2, CR, IW), jnp.int32),
        pltpu.VMEM((CE, DE), jnp.float32),
        pltpu.VMEM((CE, DE), jnp.float32),
        pltpu.SemaphoreType.DMA,
        pltpu.SemaphoreType.DMA,
    ],
)
def _scatter(dst_hbm, lab_hbm, sums0_hbm, sums1_hbm, deg0_hbm, deg1_hbm,
             sums_sh, deg_sh, zrow_v, zdeg_v, ones_v, idx_v, lab0_v, lab1_v,
             sem_in, sem_sc):
    cid = lax.axis_index("c")
    sid = lax.axis_index("s")
    w = sid * 2 + cid
    npt = NPAD // 16

    zf = jnp.zeros((16,), jnp.float32)

    def zrow_body(i, _):
        zrow_v[i, :] = zf
        return 0

    lax.fori_loop(0, npt, zrow_body, 0)
    _zero_i32(zdeg_v, npt)
    one = jnp.ones((16,), jnp.int32)

    for i in range(7):
        ones_v[pl.ds(i * 16, 16)] = one
    ones_v[pl.ds(IW - 16, 16)] = one
    pltpu.sync_copy(zrow_v, sums_sh.at[pl.ds(sid * npt, npt)])
    pltpu.sync_copy(zdeg_v, deg_sh.at[pl.ds(sid * npt, npt)])
    plsc.subcore_barrier()

    base = w * EPW
    base_r = w * (EPW // IW)
    labs = (lab0_v, lab1_v)

    def prefetch(k, b):
        return [
            pltpu.async_copy(dst_hbm.at[pl.ds(base_r + k * CR, CR)],
                             idx_v.at[b], sem_in),
            pltpu.async_copy(lab_hbm.at[pl.ds(base + k * CE, CE)],
                             labs[b], sem_in),
        ]

    def fire(b):
        cps = []
        for r in range(CR):
            cps.append(pltpu.async_copy(
                labs[b].at[pl.ds(r * IW, IW)],
                sums_sh.at[idx_v.at[b, r]], sem_sc, add=True))
            cps.append(pltpu.async_copy(
                ones_v, deg_sh.at[idx_v.at[b, r]], sem_sc, add=True))
        return cps

    pf = {0: prefetch(0, 0)}
    sc = {}
    for k in range(NCH):
        b = k % 2
        for c in pf.pop(k):
            c.wait()
        if k + 1 < NCH:
            if k - 1 >= 0:
                for c in sc.pop(k - 1):
                    c.wait()
            pf[k + 1] = prefetch(k + 1, 1 - b)
        sc[k] = fire(b)
    for k in sorted(sc):
        for c in sc[k]:
            c.wait()

    plsc.subcore_barrier()

    @pl.when(cid == 0)
    def _():
        pltpu.sync_copy(sums_sh.at[pl.ds(sid * npt, npt)],
                        sums0_hbm.at[pl.ds(sid * npt, npt)])
        pltpu.sync_copy(deg_sh.at[pl.ds(sid * npt, npt)],
                        deg0_hbm.at[pl.ds(sid * npt, npt)])

    @pl.when(cid == 1)
    def _():
        pltpu.sync_copy(sums_sh.at[pl.ds(sid * npt, npt)],
                        sums1_hbm.at[pl.ds(sid * npt, npt)])
        pltpu.sync_copy(deg_sh.at[pl.ds(sid * npt, npt)],
                        deg1_hbm.at[pl.ds(sid * npt, npt)])


FULL_U = N // 128
TAIL_R = N - FULL_U * 128


@functools.partial(
    pl.kernel,
    out_type=jax.ShapeDtypeStruct((N, DF + DE), jnp.float32),
    mesh=_mesh,
    compiler_params=pltpu.CompilerParams(use_tc_tiling_on_sc=False, needs_layout_passes=False),
    scratch_types=[
        pltpu.VMEM_SHARED((HBINS,), jnp.int32),
        pltpu.VMEM((HPT,), jnp.int32),
        pltpu.VMEM((128,), jnp.int32),
        pltpu.VMEM((NPAD // 16,), jnp.int32),
        pltpu.VMEM((NPAD // 16,), jnp.int32),
        pltpu.VMEM((NPAD // (16 * 128), 128), jnp.int32),
        pltpu.VMEM((1, 128), jnp.int32),
        pltpu.VMEM((128,), jnp.int32),
        pltpu.VMEM((128,), jnp.float32),
        pltpu.VMEM((128, DE), jnp.float32),
        pltpu.VMEM((128, DE), jnp.float32),
        pltpu.VMEM((128, DF), jnp.float32),
        pltpu.VMEM((128, DF + DE), jnp.float32),
        pltpu.SemaphoreType.DMA,
    ],
)
def _finalize(h_hbm, sums0_hbm, sums1_hbm, deg0_hbm, deg1_hbm, out_hbm,
              hist_sh, zb_v, ones_v, d0_v, d1_v, didx_v, du_v,
              bkt_v, bsz_v, s0_v, s1_v, h_v, out_v, sem):
    cid = lax.axis_index("c")
    sid = lax.axis_index("s")
    w = sid * 2 + cid
    npt = NPAD // 16

    _zero_i32(zb_v, HPT)
    one = jnp.ones((16,), jnp.int32)
    for i in range(128 // 16):
        ones_v[pl.ds(i * 16, 16)] = one
    pltpu.sync_copy(zb_v, hist_sh.at[pl.ds(sid * HPT, HPT)])
    plsc.subcore_barrier()

    pltpu.sync_copy(deg0_hbm.at[pl.ds(sid * npt, npt)], d0_v)
    pltpu.sync_copy(deg1_hbm.at[pl.ds(sid * npt, npt)], d1_v)
    nrow = npt // 128

    def deg_body(i, _):
        r = i // 8
        l = i % 8
        didx_v[r, pl.ds(l * 16, 16)] = (
            d0_v[pl.ds(i * 16, 16)] + d1_v[pl.ds(i * 16, 16)])
        return 0

    lax.fori_loop(0, npt // 16, deg_body, 0)
    cps = [pltpu.async_copy(ones_v, hist_sh.at[didx_v.at[r]], sem, add=True)
           for r in range(nrow)]
    for c in cps:
        c.wait()
    plsc.subcore_barrier()

    def elu(x):
        return jnp.where(x > 0, x, jnp.exp(x) - 1.0)

    for k in range(3):
        u = w + 32 * k

        @pl.when(u <= FULL_U)
        def _():
            base = u * 128
            pltpu.sync_copy(deg0_hbm.at[pl.ds(base, 128)], bkt_v)
            pltpu.sync_copy(deg1_hbm.at[pl.ds(base, 128)], du_v.at[0])
            for i in range(8):
                du_v[0, pl.ds(i * 16, 16)] = (
                    bkt_v[pl.ds(i * 16, 16)] + du_v[0, pl.ds(i * 16, 16)])
            pltpu.sync_copy(hist_sh.at[du_v.at[0]], bkt_v)
            for i in range(8):
                b = bkt_v[pl.ds(i * 16, 16)].astype(jnp.float32)
                bsz_v[pl.ds(i * 16, 16)] = jnp.maximum(b, 1.0)
            pltpu.sync_copy(sums0_hbm.at[pl.ds(base, 128)], s0_v)
            pltpu.sync_copy(sums1_hbm.at[pl.ds(base, 128)], s1_v)

            @pl.when(u < FULL_U)
            def _():
                pltpu.sync_copy(h_hbm.at[pl.ds(base, 128)], h_v)

            @pl.when(u == FULL_U)
            def _():
                pltpu.sync_copy(h_hbm.at[pl.ds(FULL_U * 128, TAIL_R)],
                                h_v.at[pl.ds(0, TAIL_R)])

            def row_body(i, _):
                s = s0_v[i, :] + s1_v[i, :]
                bs = plsc.load_gather(
                    bsz_v, [jnp.full((16,), i, dtype=jnp.int32)])
                out_v[i, pl.ds(DF, DE)] = elu(s / bs)
                for j in range(DF // 16):
                    x = h_v[i, pl.ds(j * 16, 16)]
                    out_v[i, pl.ds(j * 16, 16)] = elu(x)
                return 0

            lax.fori_loop(0, 128, row_body, 0)

            @pl.when(u < FULL_U)
            def _():
                pltpu.sync_copy(out_v, out_hbm.at[pl.ds(base, 128)])

            @pl.when(u == FULL_U)
            def _():
                pltpu.sync_copy(out_v.at[pl.ds(0, TAIL_R)],
                                out_hbm.at[pl.ds(FULL_U * 128, TAIL_R)])


def kernel(h, edge_index, edge_labels):
    dst = edge_index[1].astype(jnp.int32).reshape(E // IW, IW)
    sums0, sums1, deg0, deg1 = _scatter(dst, edge_labels)
    return _finalize(h, sums0, sums1, deg0, deg1)


# speedup vs baseline: 4.9578x; 1.3044x over previous
import functools

import jax
import jax.numpy as jnp
from jax import lax
from jax.experimental import pallas as pl
from jax.experimental.pallas import tpu as pltpu
from jax.experimental.pallas import tpu_sc as plsc

N = 10000
E = 320000
DF = 128
DE = 16
NPAD = 10240
EPW = E // 32
NCH = 10
IW = 125
CR = 8
CE = IW * CR
HBINS = 320256
HPT = HBINS // 16

_mesh = plsc.VectorSubcoreMesh(core_axis_name="c", subcore_axis_name="s")


def _zero_i32(ref, n):
    z = jnp.zeros((16,), jnp.int32)

    def body(i, _):
        ref[pl.ds(i * 16, 16)] = z
        return 0

    lax.fori_loop(0, n // 16, body, 0)


@functools.partial(
    pl.kernel,
    out_type=(
        jax.ShapeDtypeStruct((NPAD, DE), jnp.float32),
        jax.ShapeDtypeStruct((NPAD, DE), jnp.float32),
        jax.ShapeDtypeStruct((NPAD,), jnp.float32),
    ),
    mesh=_mesh,
    compiler_params=pltpu.CompilerParams(
        use_tc_tiling_on_sc=False, needs_layout_passes=False),
    scratch_types=[
        pltpu.VMEM_SHARED((NPAD, DE), jnp.float32),
        pltpu.VMEM_SHARED((NPAD,), jnp.int32),
        pltpu.VMEM_SHARED((HBINS,), jnp.int32),
        pltpu.VMEM((NPAD // 16, DE), jnp.float32),
        pltpu.VMEM((NPAD // 16,), jnp.int32),
        pltpu.VMEM((HPT,), jnp.int32),
        pltpu.VMEM((128,), jnp.int32),
        pltpu.VMEM((2, CR, IW), jnp.int32),
        pltpu.VMEM((2, CR, IW), jnp.int32),
        pltpu.VMEM((CE, DE), jnp.float32),
        pltpu.VMEM((CE, DE), jnp.float32),
        pltpu.VMEM((NPAD // (16 * 128), 128), jnp.int32),
        pltpu.VMEM((128,), jnp.int32),
        pltpu.VMEM((NPAD // 16,), jnp.float32),
        pltpu.SemaphoreType.DMA,
        pltpu.SemaphoreType.DMA,
    ],
)
def _fused(dst_hbm, lab_hbm, sums0_hbm, sums1_hbm, bsz_hbm,
           sums_sh, deg_sh, hist_sh, zrow_v, zdeg_v, zb_v, ones_v,
           idx_v, idx2_v, lab0_v, lab1_v, didx_v, bgt_v, bszf_v,
           sem_in, sem_sc):
    cid = lax.axis_index("c")
    sid = lax.axis_index("s")
    w = sid * 2 + cid
    w2 = sid * 2 + (1 - cid)
    npt = NPAD // 16

    zf = jnp.zeros((16,), jnp.float32)

    def zrow_body(i, _):
        zrow_v[i, :] = zf
        return 0

    lax.fori_loop(0, npt, zrow_body, 0)
    _zero_i32(zdeg_v, npt)
    _zero_i32(zb_v, HPT)
    one = jnp.ones((16,), jnp.int32)
    for i in range(8):
        ones_v[pl.ds(i * 16, 16)] = one
    pltpu.sync_copy(zrow_v, sums_sh.at[pl.ds(sid * npt, npt)])
    pltpu.sync_copy(zdeg_v, deg_sh.at[pl.ds(sid * npt, npt)])
    pltpu.sync_copy(zb_v, hist_sh.at[pl.ds(sid * HPT, HPT)])
    plsc.subcore_barrier()

    base = w * EPW
    base_r = w * (EPW // IW)
    base_r2 = w2 * (EPW // IW)
    labs = (lab0_v, lab1_v)

    def prefetch(k, b):
        return [
            pltpu.async_copy(dst_hbm.at[pl.ds(base_r + k * CR, CR)],
                             idx_v.at[b], sem_in),
            pltpu.async_copy(dst_hbm.at[pl.ds(base_r2 + k * CR, CR)],
                             idx2_v.at[b], sem_in),
            pltpu.async_copy(lab_hbm.at[pl.ds(base + k * CE, CE)],
                             labs[b], sem_in),
        ]

    def fire(b):
        cps = []
        for r in range(CR):
            cps.append(pltpu.async_copy(
                labs[b].at[pl.ds(r * IW, IW)],
                sums_sh.at[idx_v.at[b, r]], sem_sc, add=True))
            cps.append(pltpu.async_copy(
                ones_v.at[pl.ds(0, IW)], deg_sh.at[idx_v.at[b, r]],
                sem_sc, add=True))
            cps.append(pltpu.async_copy(
                ones_v.at[pl.ds(0, IW)], deg_sh.at[idx2_v.at[b, r]],
                sem_sc, add=True))
        return cps

    pf = {0: prefetch(0, 0)}
    sc = {}
    for k in range(NCH):
        b = k % 2
        for c in pf.pop(k):
            c.wait()
        if k + 1 < NCH:
            if k - 1 >= 0:
                for c in sc.pop(k - 1):
                    c.wait()
            pf[k + 1] = prefetch(k + 1, 1 - b)
        sc[k] = fire(b)
    for k in sorted(sc):
        for c in sc[k]:
            c.wait()
    plsc.subcore_barrier()

    nrow = npt // 128
    for r in range(nrow):
        pltpu.sync_copy(deg_sh.at[pl.ds(sid * npt + r * 128, 128)],
                        didx_v.at[r])
    cps = [pltpu.async_copy(ones_v, hist_sh.at[didx_v.at[r]],
                            sem_sc, add=True)
           for r in range(nrow)]
    for c in cps:
        c.wait()
    plsc.subcore_barrier()

    for r in range(nrow):
        pltpu.sync_copy(hist_sh.at[didx_v.at[r]], bgt_v)
        for i in range(8):
            bf = bgt_v[pl.ds(i * 16, 16)].astype(jnp.float32)
            bszf_v[pl.ds(r * 128 + i * 16, 16)] = jnp.maximum(bf, 1.0)

    @pl.when(cid == 0)
    def _():
        pltpu.sync_copy(sums_sh.at[pl.ds(sid * npt, npt)],
                        sums0_hbm.at[pl.ds(sid * npt, npt)])
        pltpu.sync_copy(bszf_v, bsz_hbm.at[pl.ds(sid * npt, npt)])

    @pl.when(cid == 1)
    def _():
        pltpu.sync_copy(sums_sh.at[pl.ds(sid * npt, npt)],
                        sums1_hbm.at[pl.ds(sid * npt, npt)])


R = 400


def _combine_body(h_ref, s0_ref, s1_ref, b_ref, o_ref):
    hv = h_ref[...]
    acc = (s0_ref[...] + s1_ref[...]) / b_ref[...]
    o_ref[:, :DF] = jnp.where(hv > 0, hv, jnp.exp(hv) - 1.0)
    o_ref[:, DF:] = jnp.where(acc > 0, acc, jnp.exp(acc) - 1.0)


_combine = pl.pallas_call(
    _combine_body,
    out_shape=jax.ShapeDtypeStruct((N, DF + DE), jnp.float32),
    grid=(N // R,),
    in_specs=[
        pl.BlockSpec((R, DF), lambda i: (i, 0)),
        pl.BlockSpec((R, DE), lambda i: (i, 0)),
        pl.BlockSpec((R, DE), lambda i: (i, 0)),
        pl.BlockSpec((R, 1), lambda i: (i, 0)),
    ],
    out_specs=pl.BlockSpec((R, DF + DE), lambda i: (i, 0)),
)


def kernel(h, edge_index, edge_labels):
    dst = edge_index[1].astype(jnp.int32).reshape(E // IW, IW)
    sums0, sums1, bsz = _fused(dst, edge_labels)
    return _combine(h, sums0[:N], sums1[:N], bsz[:N, None])
